# baseline (speedup 1.0000x reference)
import functools

import jax
import jax.numpy as jnp
from jax import lax
from jax.experimental import pallas as pl
from jax.experimental.pallas import tpu as pltpu
from jax.experimental.pallas import tpu_sc as plsc

_N = 10000
_E = 320000
_F = 128
_DE = 4
_L = 4
_M = 128

_NN = 10240
_W = 144
_WC = 32
_WA = 16
_WH = 64
_NC, _NS = 2, 16
_NWORK = _NC * _NS
_K = 128
_NB = 4
_CHUNKS = 80
_EPW = _K * _CHUNKS
_EP = _NWORK * _EPW
_NCH = _EP // _K

_BE = 4096
_BN = 1280

_ROWS_PER_TILE = _NN // _NS
_ZCH = _ROWS_PER_TILE // _K


def _sc_gather_body(featsg_hbm, coorsg_hbm, row_hbm, col_hbm,
                    frow_hbm, fcol_hbm, crow_hbm, ccol_hbm,
                    idxr_v, idxc_v, fr_v, fc_v, cr_v, cc_v, semg, semw):
    c = lax.axis_index("c")
    s = lax.axis_index("s")
    wid = s * _NC + c
    base = wid * (_CHUNKS // _NB)

    def batch(t, carry):
        c0 = (base + t) * _NB
        e0 = c0 * _K
        pltpu.sync_copy(row_hbm.at[pl.ds(c0, _NB)], idxr_v)
        pltpu.sync_copy(col_hbm.at[pl.ds(c0, _NB)], idxc_v)
        gd = []
        for b in range(_NB):
            gd.append(pltpu.async_copy(
                featsg_hbm.at[idxr_v.at[b]], fr_v.at[b], semg))
            gd.append(pltpu.async_copy(
                featsg_hbm.at[idxc_v.at[b]], fc_v.at[b], semg))
            gd.append(pltpu.async_copy(
                coorsg_hbm.at[idxr_v.at[b]], cr_v.at[b], semg))
            gd.append(pltpu.async_copy(
                coorsg_hbm.at[idxc_v.at[b]], cc_v.at[b], semg))
        for d in gd:
            d.wait()
        wd = []
        for b in range(_NB):
            wd.append(pltpu.async_copy(
                fr_v.at[b], frow_hbm.at[pl.ds(e0 + b * _K, _K)], semw))
            wd.append(pltpu.async_copy(
                fc_v.at[b], fcol_hbm.at[pl.ds(e0 + b * _K, _K)], semw))
            wd.append(pltpu.async_copy(
                cr_v.at[b], crow_hbm.at[pl.ds(e0 + b * _K, _K)], semw))
            wd.append(pltpu.async_copy(
                cc_v.at[b], ccol_hbm.at[pl.ds(e0 + b * _K, _K)], semw))
        for d in wd:
            d.wait()
        return carry

    lax.fori_loop(0, _CHUNKS // _NB, batch, 0)


@functools.lru_cache(maxsize=None)
def _sc_gather_kernel():
    mesh = plsc.VectorSubcoreMesh(core_axis_name="c", subcore_axis_name="s")
    return pl.kernel(
        _sc_gather_body,
        out_type=(jax.ShapeDtypeStruct((_EP, _F), jnp.bfloat16),
                  jax.ShapeDtypeStruct((_EP, _F), jnp.bfloat16),
                  jax.ShapeDtypeStruct((_EP, _WC), jnp.bfloat16),
                  jax.ShapeDtypeStruct((_EP, _WC), jnp.bfloat16)),
        mesh=mesh,
        scratch_types=[
            pltpu.VMEM((_NB, _K), jnp.int32),
            pltpu.VMEM((_NB, _K), jnp.int32),
            pltpu.VMEM((_NB, _K, _F), jnp.bfloat16),
            pltpu.VMEM((_NB, _K, _F), jnp.bfloat16),
            pltpu.VMEM((_NB, _K, _WC), jnp.bfloat16),
            pltpu.VMEM((_NB, _K, _WC), jnp.bfloat16),
            pltpu.SemaphoreType.DMA,
            pltpu.SemaphoreType.DMA,
        ],
        compiler_params=pltpu.CompilerParams(use_tc_tiling_on_sc=False),
    )


def _sc_gather(featsg, coorsg, row, col):
    return _sc_gather_kernel()(featsg, coorsg, row, col)


def _sc_scatter_body(m_hbm, aux_hbm, col_hbm, outm_hbm, outa_hbm,
                     idx0_v, idx1_v, vm0_v, vm1_v, va0_v, va1_v,
                     zb_v, accm_sh, acca_sh, sem):
    c = lax.axis_index("c")
    s = lax.axis_index("s")
    lb = c * _WH

    def zrow(j, carry):
        for q in range(_WH // 16):
            zb_v[j, pl.ds(q * 16, 16)] = jnp.zeros((16,), jnp.float32)
        return carry

    lax.fori_loop(0, _K, zrow, 0)
    for t in range(_ZCH):
        pltpu.sync_copy(zb_v, accm_sh.at[pl.ds(s * _ROWS_PER_TILE + t * _K, _K)])
    for t in range(_ZCH):
        pltpu.sync_copy(zb_v.at[pl.ds(0, _K), pl.ds(0, _WA)],
                        acca_sh.at[pl.ds(s * _ROWS_PER_TILE + t * _K, _K)])
    plsc.subcore_barrier()

    base = s * (_NCH // _NS)

    def batch(i, carry):
        c0 = base + i * 2
        e0 = c0 * _K
        gd = [pltpu.async_copy(col_hbm.at[pl.ds(e0, _K)], idx0_v, sem),
              pltpu.async_copy(m_hbm.at[pl.ds(e0, _K), pl.ds(lb, _WH)],
                               vm0_v, sem),
              pltpu.async_copy(col_hbm.at[pl.ds(e0 + _K, _K)], idx1_v, sem),
              pltpu.async_copy(m_hbm.at[pl.ds(e0 + _K, _K), pl.ds(lb, _WH)],
                               vm1_v, sem)]
        for d in gd:
            d.wait()
        pltpu.sync_copy(vm0_v, accm_sh.at[idx0_v], add=True)
        pltpu.sync_copy(vm1_v, accm_sh.at[idx1_v], add=True)

        @pl.when(c == 0)
        def _aux():
            d0 = pltpu.async_copy(aux_hbm.at[pl.ds(e0, _K)], va0_v, sem)
            d1 = pltpu.async_copy(aux_hbm.at[pl.ds(e0 + _K, _K)], va1_v, sem)
            d0.wait()
            d1.wait()
            pltpu.sync_copy(va0_v, acca_sh.at[idx0_v], add=True)
            pltpu.sync_copy(va1_v, acca_sh.at[idx1_v], add=True)

        return carry

    lax.fori_loop(0, _NCH // _NS // 2, batch, 0)
    plsc.subcore_barrier()

    for t in range(_ZCH):
        r0 = s * _ROWS_PER_TILE + t * _K
        pltpu.sync_copy(accm_sh.at[pl.ds(r0, _K)], zb_v)
        pltpu.sync_copy(zb_v, outm_hbm.at[pl.ds(r0, _K), pl.ds(lb, _WH)])

    @pl.when(c == 0)
    def _draina():
        for t in range(_ZCH):
            r0 = s * _ROWS_PER_TILE + t * _K
            pltpu.sync_copy(acca_sh.at[pl.ds(r0, _K)],
                            zb_v.at[pl.ds(0, _K), pl.ds(0, _WA)])
            pltpu.sync_copy(zb_v.at[pl.ds(0, _K), pl.ds(0, _WA)],
                            outa_hbm.at[pl.ds(r0, _K)])


@functools.lru_cache(maxsize=None)
def _sc_scatter_kernel():
    mesh = plsc.VectorSubcoreMesh(core_axis_name="c", subcore_axis_name="s")
    return pl.kernel(
        _sc_scatter_body,
        out_type=(jax.ShapeDtypeStruct((_NN, _F), jnp.float32),
                  jax.ShapeDtypeStruct((_NN, _WA), jnp.float32)),
        mesh=mesh,
        scratch_types=[
            pltpu.VMEM((_K,), jnp.int32),
            pltpu.VMEM((_K,), jnp.int32),
            pltpu.VMEM((_K, _WH), jnp.float32),
            pltpu.VMEM((_K, _WH), jnp.float32),
            pltpu.VMEM((_K, _WA), jnp.float32),
            pltpu.VMEM((_K, _WA), jnp.float32),
            pltpu.VMEM((_K, _WH), jnp.float32),
            pltpu.VMEM_SHARED((_NN, _WH), jnp.float32),
            pltpu.VMEM_SHARED((_NN, _WA), jnp.float32),
            pltpu.SemaphoreType.DMA,
        ],
        compiler_params=pltpu.CompilerParams(use_tc_tiling_on_sc=False),
    )


def _sc_scatter(m, aux, col):
    return _sc_scatter_kernel()(m, aux, col)


def _silu(x):
    return x * jax.nn.sigmoid(x)


def _edge_body(ea_ref, frow_ref, fcol_ref, crow_ref, ccol_ref,
               w1r_ref, w1c_ref, w1rd_ref, w1ea_ref, b1_ref, w2_ref, b2_ref,
               wc1_ref, bc1_ref, wc2_ref, bc2_ref, outm_ref, outa_ref):
    f32 = jnp.float32
    bf16 = jnp.bfloat16
    fr = frow_ref[...]
    fc = fcol_ref[...]
    geo = (crow_ref[...].astype(f32) - ccol_ref[...].astype(f32))
    rd = jnp.sum(geo * geo, axis=1, keepdims=True)
    ea = ea_ref[...]
    h = (jnp.dot(fr, w1r_ref[...], preferred_element_type=f32)
         + jnp.dot(fc, w1c_ref[...], preferred_element_type=f32)
         + rd * w1rd_ref[...]
         + jnp.dot(ea, w1ea_ref[...], preferred_element_type=f32)
         + b1_ref[...])
    h = _silu(h)
    m = _silu(jnp.dot(h.astype(bf16), w2_ref[...],
                      preferred_element_type=f32) + b2_ref[...])
    c1 = _silu(jnp.dot(m.astype(bf16), wc1_ref[...],
                       preferred_element_type=f32) + bc1_ref[...])
    w = jnp.sum(c1 * wc2_ref[...], axis=1, keepdims=True) + bc2_ref[...]
    relw = geo[:, :_WA] * w
    lane = lax.broadcasted_iota(jnp.int32, relw.shape, 1)
    relw = jnp.where(lane == _WA - 1, 1.0, relw)
    outm_ref[...] = m
    outa_ref[...] = relw


def _edge_call(ea, frow, fcol, crow, ccol, w1r, w1c, w1rd, w1ea, b1, w2, b2,
               wc1, bc1, wc2, bc2):
    grid = _EP // _BE
    full = lambda i: (0, 0)
    return pl.pallas_call(
        _edge_body,
        grid=(grid,),
        in_specs=[
            pl.BlockSpec((_BE, 8), lambda i: (i, 0)),
            pl.BlockSpec((_BE, _F), lambda i: (i, 0)),
            pl.BlockSpec((_BE, _F), lambda i: (i, 0)),
            pl.BlockSpec((_BE, _WC), lambda i: (i, 0)),
            pl.BlockSpec((_BE, _WC), lambda i: (i, 0)),
            pl.BlockSpec((_F, _M), full),
            pl.BlockSpec((_F, _M), full),
            pl.BlockSpec((1, _M), full),
            pl.BlockSpec((8, _M), full),
            pl.BlockSpec((1, _M), full),
            pl.BlockSpec((_M, _M), full),
            pl.BlockSpec((1, _M), full),
            pl.BlockSpec((_M, 4 * _M), full),
            pl.BlockSpec((1, 4 * _M), full),
            pl.BlockSpec((1, 4 * _M), full),
            pl.BlockSpec((1, 1), full),
        ],
        out_specs=(pl.BlockSpec((_BE, _F), lambda i: (i, 0)),
                   pl.BlockSpec((_BE, _WA), lambda i: (i, 0))),
        out_shape=(jax.ShapeDtypeStruct((_EP, _F), jnp.float32),
                   jax.ShapeDtypeStruct((_EP, _WA), jnp.float32)),
        compiler_params=pltpu.CompilerParams(
            dimension_semantics=("arbitrary",)),
    )(ea, frow, fcol, crow, ccol, w1r, w1c, w1rd, w1ea, b1, w2, b2,
      wc1, bc1, wc2, bc2)


def _node_body(table_ref, pm_ref, pa_ref, wn1a_ref, wn1b_ref, bn1_ref,
               wn2_ref, bn2_ref, out_ref, outf_ref, outc_ref):
    f32 = jnp.float32
    bf16 = jnp.bfloat16
    t = table_ref[...]
    feats = t[:, :_F]
    coor16 = t[:, _F:_W]
    m_i = pm_ref[...]
    upd = pa_ref[...]
    cnt = upd[:, _WA - 1:_WA]
    h = _silu(jnp.dot(feats.astype(bf16), wn1a_ref[...],
                      preferred_element_type=f32)
              + jnp.dot(m_i.astype(bf16), wn1b_ref[...],
                        preferred_element_type=f32)
              + bn1_ref[...])
    fnew = feats + jnp.dot(h.astype(bf16), wn2_ref[...],
                           preferred_element_type=f32) + bn2_ref[...]
    inv = 1.0 / jnp.maximum(cnt, 1.0)
    lane = lax.broadcasted_iota(jnp.int32, upd.shape, 1)
    cnew16 = coor16 + jnp.where(lane < 3, upd * inv, 0.0)
    out_ref[:, :_F] = fnew
    out_ref[:, _F:_W] = cnew16
    outf_ref[...] = fnew.astype(bf16)
    outc_ref[:, :_WA] = cnew16.astype(bf16)
    outc_ref[:, _WA:_WC] = jnp.zeros((fnew.shape[0], _WC - _WA), bf16)


def _node_call(table, pm, pa, wn1a, wn1b, bn1, wn2, bn2):
    grid = _NN // _BN
    full = lambda i: (0, 0)
    return pl.pallas_call(
        _node_body,
        grid=(grid,),
        in_specs=[
            pl.BlockSpec((_BN, _W), lambda i: (i, 0)),
            pl.BlockSpec((_BN, _F), lambda i: (i, 0)),
            pl.BlockSpec((_BN, _WA), lambda i: (i, 0)),
            pl.BlockSpec((_F, 2 * _M), full),
            pl.BlockSpec((_M, 2 * _M), full),
            pl.BlockSpec((1, 2 * _M), full),
            pl.BlockSpec((2 * _M, _F), full),
            pl.BlockSpec((1, _F), full),
        ],
        out_specs=(pl.BlockSpec((_BN, _W), lambda i: (i, 0)),
                   pl.BlockSpec((_BN, _F), lambda i: (i, 0)),
                   pl.BlockSpec((_BN, _WC), lambda i: (i, 0))),
        out_shape=(jax.ShapeDtypeStruct((_NN, _W), jnp.float32),
                   jax.ShapeDtypeStruct((_NN, _F), jnp.bfloat16),
                   jax.ShapeDtypeStruct((_NN, _WC), jnp.bfloat16)),
        compiler_params=pltpu.CompilerParams(
            dimension_semantics=("arbitrary",)),
    )(table, pm, pa, wn1a, wn1b, bn1, wn2, bn2)


def kernel(x, pos, edges, edge_attr, W1, b1, W2, b2, Wc1, bc1, Wc2, bc2,
           Wn1, bn1, Wn2, bn2):
    f32 = jnp.float32
    bf16 = jnp.bfloat16
    row = jnp.pad(edges[0], (0, _EP - _E), constant_values=_N).astype(jnp.int32)
    col = jnp.pad(edges[1], (0, _EP - _E), constant_values=_N).astype(jnp.int32)
    row2 = row.reshape(_NCH, _K)
    col2 = col.reshape(_NCH, _K)
    ea = jnp.pad(edge_attr, ((0, _EP - _E), (0, 8 - _DE))).astype(bf16)

    feats_p = jnp.pad(x, ((0, _NN - _N), (0, 0)))
    coors_p = jnp.pad(pos, ((0, _NN - _N), (0, _W - _F - 3)))
    table = jnp.concatenate([feats_p, coors_p], axis=1)
    featsg = feats_p.astype(bf16)
    coorsg = jnp.pad(pos, ((0, _NN - _N), (0, _WC - 3))).astype(bf16)

    for l in range(_L):
        w1 = W1[l].astype(bf16)
        w1r = w1[:_F]
        w1c = w1[_F:2 * _F]
        w1rd = W1[l][2 * _F:2 * _F + 1]
        w1ea = jnp.pad(w1[2 * _F + 1:], ((0, 4), (0, 0)))
        frow, fcol, crow, ccol = _sc_gather(featsg, coorsg, row2, col2)
        m, aux = _edge_call(ea, frow, fcol, crow, ccol, w1r, w1c, w1rd, w1ea,
                            b1[l][None], W2[l].astype(bf16), b2[l][None],
                            Wc1[l].astype(bf16), bc1[l][None],
                            Wc2[l].T, bc2[l][None])
        pm, pa = _sc_scatter(m, aux, col)
        table, featsg, coorsg = _node_call(table, pm, pa,
                                           Wn1[l][:_F].astype(bf16),
                                           Wn1[l][_F:].astype(bf16),
                                           bn1[l][None],
                                           Wn2[l].astype(bf16),
                                           bn2[l][None])

    return jnp.concatenate([table[:_N, _F:_F + 3], table[:_N, :_F]], axis=1)


# speedup vs baseline: 1.0261x; 1.0261x over previous
import functools

import jax
import jax.numpy as jnp
from jax import lax
from jax.experimental import pallas as pl
from jax.experimental.pallas import tpu as pltpu
from jax.experimental.pallas import tpu_sc as plsc

_N = 10000
_E = 320000
_F = 128
_DE = 4
_L = 4
_M = 128

_NN = 10240
_W = 144
_WC = 32
_WA = 16
_WH = 64
_NC, _NS = 2, 16
_NWORK = _NC * _NS
_K = 128
_NB = 4
_CHUNKS = 80
_EPW = _K * _CHUNKS
_EP = _NWORK * _EPW
_NCH = _EP // _K

_BE = 2048
_BN = 1280

_ROWS_PER_TILE = _NN // _NS
_ZCH = _ROWS_PER_TILE // _K


def _sc_gather_body(featsg_hbm, coorsg_hbm, row_hbm, col_hbm,
                    frow_hbm, fcol_hbm, crow_hbm, ccol_hbm,
                    idxr_v, idxc_v, fr_v, fc_v, cr_v, cc_v, semg, semw):
    c = lax.axis_index("c")
    s = lax.axis_index("s")
    wid = s * _NC + c
    base = wid * (_CHUNKS // _NB)

    def batch(t, carry):
        c0 = (base + t) * _NB
        e0 = c0 * _K
        pltpu.sync_copy(row_hbm.at[pl.ds(c0, _NB)], idxr_v)
        pltpu.sync_copy(col_hbm.at[pl.ds(c0, _NB)], idxc_v)
        gd = []
        for b in range(_NB):
            gd.append(pltpu.async_copy(
                featsg_hbm.at[idxr_v.at[b]], fr_v.at[b], semg))
            gd.append(pltpu.async_copy(
                featsg_hbm.at[idxc_v.at[b]], fc_v.at[b], semg))
            gd.append(pltpu.async_copy(
                coorsg_hbm.at[idxr_v.at[b]], cr_v.at[b], semg))
            gd.append(pltpu.async_copy(
                coorsg_hbm.at[idxc_v.at[b]], cc_v.at[b], semg))
        for d in gd:
            d.wait()
        wd = []
        for b in range(_NB):
            wd.append(pltpu.async_copy(
                fr_v.at[b], frow_hbm.at[pl.ds(e0 + b * _K, _K)], semw))
            wd.append(pltpu.async_copy(
                fc_v.at[b], fcol_hbm.at[pl.ds(e0 + b * _K, _K)], semw))
            wd.append(pltpu.async_copy(
                cr_v.at[b], crow_hbm.at[pl.ds(e0 + b * _K, _K)], semw))
            wd.append(pltpu.async_copy(
                cc_v.at[b], ccol_hbm.at[pl.ds(e0 + b * _K, _K)], semw))
        for d in wd:
            d.wait()
        return carry

    lax.fori_loop(0, _CHUNKS // _NB, batch, 0)


@functools.lru_cache(maxsize=None)
def _sc_gather_kernel():
    mesh = plsc.VectorSubcoreMesh(core_axis_name="c", subcore_axis_name="s")
    return pl.kernel(
        _sc_gather_body,
        out_type=(jax.ShapeDtypeStruct((_EP, _F), jnp.bfloat16),
                  jax.ShapeDtypeStruct((_EP, _F), jnp.bfloat16),
                  jax.ShapeDtypeStruct((_EP, _WC), jnp.bfloat16),
                  jax.ShapeDtypeStruct((_EP, _WC), jnp.bfloat16)),
        mesh=mesh,
        scratch_types=[
            pltpu.VMEM((_NB, _K), jnp.int32),
            pltpu.VMEM((_NB, _K), jnp.int32),
            pltpu.VMEM((_NB, _K, _F), jnp.bfloat16),
            pltpu.VMEM((_NB, _K, _F), jnp.bfloat16),
            pltpu.VMEM((_NB, _K, _WC), jnp.bfloat16),
            pltpu.VMEM((_NB, _K, _WC), jnp.bfloat16),
            pltpu.SemaphoreType.DMA,
            pltpu.SemaphoreType.DMA,
        ],
        compiler_params=pltpu.CompilerParams(use_tc_tiling_on_sc=False),
    )


def _sc_gather(featsg, coorsg, row, col):
    return _sc_gather_kernel()(featsg, coorsg, row, col)


def _sc_scatter_body(m_hbm, aux_hbm, col_hbm, outm_hbm, outa_hbm,
                     idx0_v, idx1_v, vm0_v, vm1_v, va0_v, va1_v,
                     zb_v, accm_sh, acca_sh, sem):
    c = lax.axis_index("c")
    s = lax.axis_index("s")
    lb = c * _WH

    def zrow(j, carry):
        for q in range(_WH // 16):
            zb_v[j, pl.ds(q * 16, 16)] = jnp.zeros((16,), jnp.float32)
        return carry

    lax.fori_loop(0, _K, zrow, 0)
    for t in range(_ZCH):
        pltpu.sync_copy(zb_v, accm_sh.at[pl.ds(s * _ROWS_PER_TILE + t * _K, _K)])
    for t in range(_ZCH):
        pltpu.sync_copy(zb_v.at[pl.ds(0, _K), pl.ds(0, _WA)],
                        acca_sh.at[pl.ds(s * _ROWS_PER_TILE + t * _K, _K)])
    plsc.subcore_barrier()

    base = s * (_NCH // _NS)

    def batch(i, carry):
        c0 = base + i * 2
        e0 = c0 * _K
        gd = [pltpu.async_copy(col_hbm.at[pl.ds(e0, _K)], idx0_v, sem),
              pltpu.async_copy(m_hbm.at[pl.ds(e0, _K), pl.ds(lb, _WH)],
                               vm0_v, sem),
              pltpu.async_copy(col_hbm.at[pl.ds(e0 + _K, _K)], idx1_v, sem),
              pltpu.async_copy(m_hbm.at[pl.ds(e0 + _K, _K), pl.ds(lb, _WH)],
                               vm1_v, sem),
              pltpu.async_copy(aux_hbm.at[pl.ds(e0, _K)], va0_v, sem),
              pltpu.async_copy(aux_hbm.at[pl.ds(e0 + _K, _K)], va1_v, sem)]
        for d in gd:
            d.wait()
        pltpu.sync_copy(vm0_v, accm_sh.at[idx0_v], add=True)
        pltpu.sync_copy(vm1_v, accm_sh.at[idx1_v], add=True)

        @pl.when(c == 0)
        def _aux():
            pltpu.sync_copy(va0_v, acca_sh.at[idx0_v], add=True)
            pltpu.sync_copy(va1_v, acca_sh.at[idx1_v], add=True)

        return carry

    lax.fori_loop(0, _NCH // _NS // 2, batch, 0)
    plsc.subcore_barrier()

    for t in range(_ZCH):
        r0 = s * _ROWS_PER_TILE + t * _K
        pltpu.sync_copy(accm_sh.at[pl.ds(r0, _K)], zb_v)
        pltpu.sync_copy(zb_v, outm_hbm.at[pl.ds(r0, _K), pl.ds(lb, _WH)])

    @pl.when(c == 0)
    def _draina():
        for t in range(_ZCH):
            r0 = s * _ROWS_PER_TILE + t * _K
            pltpu.sync_copy(acca_sh.at[pl.ds(r0, _K)],
                            zb_v.at[pl.ds(0, _K), pl.ds(0, _WA)])
            pltpu.sync_copy(zb_v.at[pl.ds(0, _K), pl.ds(0, _WA)],
                            outa_hbm.at[pl.ds(r0, _K)])


@functools.lru_cache(maxsize=None)
def _sc_scatter_kernel():
    mesh = plsc.VectorSubcoreMesh(core_axis_name="c", subcore_axis_name="s")
    return pl.kernel(
        _sc_scatter_body,
        out_type=(jax.ShapeDtypeStruct((_NN, _F), jnp.float32),
                  jax.ShapeDtypeStruct((_NN, _WA), jnp.float32)),
        mesh=mesh,
        scratch_types=[
            pltpu.VMEM((_K,), jnp.int32),
            pltpu.VMEM((_K,), jnp.int32),
            pltpu.VMEM((_K, _WH), jnp.float32),
            pltpu.VMEM((_K, _WH), jnp.float32),
            pltpu.VMEM((_K, _WA), jnp.float32),
            pltpu.VMEM((_K, _WA), jnp.float32),
            pltpu.VMEM((_K, _WH), jnp.float32),
            pltpu.VMEM_SHARED((_NN, _WH), jnp.float32),
            pltpu.VMEM_SHARED((_NN, _WA), jnp.float32),
            pltpu.SemaphoreType.DMA,
        ],
        compiler_params=pltpu.CompilerParams(use_tc_tiling_on_sc=False),
    )


def _sc_scatter(m, aux, col):
    return _sc_scatter_kernel()(m, aux, col)


def _silu(x):
    return x * jax.nn.sigmoid(x)


def _edge_body(ea_ref, frow_ref, fcol_ref, crow_ref, ccol_ref,
               w1r_ref, w1c_ref, w1rd_ref, w1ea_ref, b1_ref, w2_ref, b2_ref,
               wc1_ref, bc1_ref, wc2_ref, bc2_ref, outm_ref, outa_ref):
    f32 = jnp.float32
    bf16 = jnp.bfloat16
    fr = frow_ref[...]
    fc = fcol_ref[...]
    geo = (crow_ref[...].astype(f32) - ccol_ref[...].astype(f32))
    rd = jnp.sum(geo * geo, axis=1, keepdims=True)
    ea = ea_ref[...]
    h = (jnp.dot(fr, w1r_ref[...], preferred_element_type=f32)
         + jnp.dot(fc, w1c_ref[...], preferred_element_type=f32)
         + rd * w1rd_ref[...]
         + jnp.dot(ea, w1ea_ref[...], preferred_element_type=f32)
         + b1_ref[...])
    h = _silu(h)
    m = _silu(jnp.dot(h.astype(bf16), w2_ref[...],
                      preferred_element_type=f32) + b2_ref[...])
    c1 = _silu(jnp.dot(m.astype(bf16), wc1_ref[...],
                       preferred_element_type=f32) + bc1_ref[...])
    w = jnp.sum(c1 * wc2_ref[...], axis=1, keepdims=True) + bc2_ref[...]
    relw = geo[:, :_WA] * w
    lane = lax.broadcasted_iota(jnp.int32, relw.shape, 1)
    relw = jnp.where(lane == _WA - 1, 1.0, relw)
    outm_ref[...] = m
    outa_ref[...] = relw


def _edge_call(ea, frow, fcol, crow, ccol, w1r, w1c, w1rd, w1ea, b1, w2, b2,
               wc1, bc1, wc2, bc2):
    grid = _EP // _BE
    full = lambda i: (0, 0)
    return pl.pallas_call(
        _edge_body,
        grid=(grid,),
        in_specs=[
            pl.BlockSpec((_BE, 8), lambda i: (i, 0)),
            pl.BlockSpec((_BE, _F), lambda i: (i, 0)),
            pl.BlockSpec((_BE, _F), lambda i: (i, 0)),
            pl.BlockSpec((_BE, _WC), lambda i: (i, 0)),
            pl.BlockSpec((_BE, _WC), lambda i: (i, 0)),
            pl.BlockSpec((_F, _M), full),
            pl.BlockSpec((_F, _M), full),
            pl.BlockSpec((1, _M), full),
            pl.BlockSpec((8, _M), full),
            pl.BlockSpec((1, _M), full),
            pl.BlockSpec((_M, _M), full),
            pl.BlockSpec((1, _M), full),
            pl.BlockSpec((_M, 4 * _M), full),
            pl.BlockSpec((1, 4 * _M), full),
            pl.BlockSpec((1, 4 * _M), full),
            pl.BlockSpec((1, 1), full),
        ],
        out_specs=(pl.BlockSpec((_BE, _F), lambda i: (i, 0)),
                   pl.BlockSpec((_BE, _WA), lambda i: (i, 0))),
        out_shape=(jax.ShapeDtypeStruct((_EP, _F), jnp.float32),
                   jax.ShapeDtypeStruct((_EP, _WA), jnp.float32)),
        compiler_params=pltpu.CompilerParams(
            dimension_semantics=("arbitrary",)),
    )(ea, frow, fcol, crow, ccol, w1r, w1c, w1rd, w1ea, b1, w2, b2,
      wc1, bc1, wc2, bc2)


def _node_body(table_ref, pm_ref, pa_ref, wn1a_ref, wn1b_ref, bn1_ref,
               wn2_ref, bn2_ref, out_ref, outf_ref, outc_ref):
    f32 = jnp.float32
    bf16 = jnp.bfloat16
    t = table_ref[...]
    feats = t[:, :_F]
    coor16 = t[:, _F:_W]
    m_i = pm_ref[...]
    upd = pa_ref[...]
    cnt = upd[:, _WA - 1:_WA]
    h = _silu(jnp.dot(feats.astype(bf16), wn1a_ref[...],
                      preferred_element_type=f32)
              + jnp.dot(m_i.astype(bf16), wn1b_ref[...],
                        preferred_element_type=f32)
              + bn1_ref[...])
    fnew = feats + jnp.dot(h.astype(bf16), wn2_ref[...],
                           preferred_element_type=f32) + bn2_ref[...]
    inv = 1.0 / jnp.maximum(cnt, 1.0)
    lane = lax.broadcasted_iota(jnp.int32, upd.shape, 1)
    cnew16 = coor16 + jnp.where(lane < 3, upd * inv, 0.0)
    out_ref[:, :_F] = fnew
    out_ref[:, _F:_W] = cnew16
    outf_ref[...] = fnew.astype(bf16)
    outc_ref[:, :_WA] = cnew16.astype(bf16)
    outc_ref[:, _WA:_WC] = jnp.zeros((fnew.shape[0], _WC - _WA), bf16)


def _node_call(table, pm, pa, wn1a, wn1b, bn1, wn2, bn2):
    grid = _NN // _BN
    full = lambda i: (0, 0)
    return pl.pallas_call(
        _node_body,
        grid=(grid,),
        in_specs=[
            pl.BlockSpec((_BN, _W), lambda i: (i, 0)),
            pl.BlockSpec((_BN, _F), lambda i: (i, 0)),
            pl.BlockSpec((_BN, _WA), lambda i: (i, 0)),
            pl.BlockSpec((_F, 2 * _M), full),
            pl.BlockSpec((_M, 2 * _M), full),
            pl.BlockSpec((1, 2 * _M), full),
            pl.BlockSpec((2 * _M, _F), full),
            pl.BlockSpec((1, _F), full),
        ],
        out_specs=(pl.BlockSpec((_BN, _W), lambda i: (i, 0)),
                   pl.BlockSpec((_BN, _F), lambda i: (i, 0)),
                   pl.BlockSpec((_BN, _WC), lambda i: (i, 0))),
        out_shape=(jax.ShapeDtypeStruct((_NN, _W), jnp.float32),
                   jax.ShapeDtypeStruct((_NN, _F), jnp.bfloat16),
                   jax.ShapeDtypeStruct((_NN, _WC), jnp.bfloat16)),
        compiler_params=pltpu.CompilerParams(
            dimension_semantics=("arbitrary",)),
    )(table, pm, pa, wn1a, wn1b, bn1, wn2, bn2)


def kernel(x, pos, edges, edge_attr, W1, b1, W2, b2, Wc1, bc1, Wc2, bc2,
           Wn1, bn1, Wn2, bn2):
    f32 = jnp.float32
    bf16 = jnp.bfloat16
    row = jnp.pad(edges[0], (0, _EP - _E), constant_values=_N).astype(jnp.int32)
    col = jnp.pad(edges[1], (0, _EP - _E), constant_values=_N).astype(jnp.int32)
    row2 = row.reshape(_NCH, _K)
    col2 = col.reshape(_NCH, _K)
    ea = jnp.pad(edge_attr, ((0, _EP - _E), (0, 8 - _DE))).astype(bf16)

    feats_p = jnp.pad(x, ((0, _NN - _N), (0, 0)))
    coors_p = jnp.pad(pos, ((0, _NN - _N), (0, _W - _F - 3)))
    table = jnp.concatenate([feats_p, coors_p], axis=1)
    featsg = feats_p.astype(bf16)
    coorsg = jnp.pad(pos, ((0, _NN - _N), (0, _WC - 3))).astype(bf16)

    for l in range(_L):
        w1 = W1[l].astype(bf16)
        w1r = w1[:_F]
        w1c = w1[_F:2 * _F]
        w1rd = W1[l][2 * _F:2 * _F + 1]
        w1ea = jnp.pad(w1[2 * _F + 1:], ((0, 4), (0, 0)))
        frow, fcol, crow, ccol = _sc_gather(featsg, coorsg, row2, col2)
        m, aux = _edge_call(ea, frow, fcol, crow, ccol, w1r, w1c, w1rd, w1ea,
                            b1[l][None], W2[l].astype(bf16), b2[l][None],
                            Wc1[l].astype(bf16), bc1[l][None],
                            Wc2[l].T, bc2[l][None])
        pm, pa = _sc_scatter(m, aux, col)
        table, featsg, coorsg = _node_call(table, pm, pa,
                                           Wn1[l][:_F].astype(bf16),
                                           Wn1[l][_F:].astype(bf16),
                                           bn1[l][None],
                                           Wn2[l].astype(bf16),
                                           bn2[l][None])

    return jnp.concatenate([table[:_N, _F:_F + 3], table[:_N, :_F]], axis=1)
